# Initial kernel scaffold; baseline (speedup 1.0000x reference)
import jax
import jax.numpy as jnp
from jax.experimental import pallas as pl


def kernel(x, w1, w2):
    raise NotImplementedError("write your pallas kernel here")


# speedup vs baseline: 2.4373x; 2.4373x over previous
import functools

import jax
import jax.numpy as jnp
from jax.experimental import pallas as pl
from jax.experimental.pallas import tpu as pltpu


def _round_up(x: int, m: int) -> int:
    return ((x + m - 1) // m) * m


def _se_kernel(x_ref, w1t_ref, w2t_ref, o_ref, *, inv_hw):
    xb = x_ref[0].astype(jnp.float32)
    pooled = jnp.sum(xb, axis=-1, keepdims=True) * inv_hw
    h = jnp.dot(w1t_ref[...], pooled,
                preferred_element_type=jnp.float32)
    h = jnp.maximum(h, 0.0)
    z = jnp.dot(w2t_ref[...], h,
                preferred_element_type=jnp.float32)
    g = jax.nn.sigmoid(z)
    o_ref[0] = (xb * g).astype(o_ref.dtype)


def kernel(x, w1, w2):
    B, C, H, W = x.shape
    HW = H * W
    rd = w1.shape[1]

    c_pad = _round_up(C, 8)
    hw_pad = _round_up(HW, 128)
    rd_pad = _round_up(rd, 8)

    x3 = x.reshape(B, C, HW)
    if c_pad != C or hw_pad != HW:
        x3 = jnp.pad(x3, ((0, 0), (0, c_pad - C), (0, hw_pad - HW)))

    w1t = w1.astype(jnp.float32).T
    w2t = w2.astype(jnp.float32).T
    if c_pad != C or rd_pad != rd:
        w1t = jnp.pad(w1t, ((0, rd_pad - rd), (0, c_pad - C)))
        w2t = jnp.pad(w2t, ((0, c_pad - C), (0, rd_pad - rd)))

    out = pl.pallas_call(
        functools.partial(_se_kernel, inv_hw=1.0 / HW),
        out_shape=jax.ShapeDtypeStruct((B, c_pad, hw_pad), x.dtype),
        grid=(B,),
        in_specs=[
            pl.BlockSpec((1, c_pad, hw_pad), lambda b: (b, 0, 0)),
            pl.BlockSpec((rd_pad, c_pad), lambda b: (0, 0)),
            pl.BlockSpec((c_pad, rd_pad), lambda b: (0, 0)),
        ],
        out_specs=pl.BlockSpec((1, c_pad, hw_pad), lambda b: (b, 0, 0)),
        compiler_params=pltpu.CompilerParams(
            dimension_semantics=("parallel",),
            vmem_limit_bytes=64 * 1024 * 1024,
        ),
    )(x3, w1t, w2t)

    if c_pad != C or hw_pad != HW:
        out = out[:, :C, :HW]
    return out.reshape(B, C, H, W)


# speedup vs baseline: 2.6013x; 1.0673x over previous
import functools

import jax
import jax.numpy as jnp
from jax.experimental import pallas as pl
from jax.experimental.pallas import tpu as pltpu


def _round_up(x: int, m: int) -> int:
    return ((x + m - 1) // m) * m


def _se_kernel(x_ref, w1t_ref, w2t_ref, o_ref, *, inv_hw, nb):
    for j in range(nb):
        xb = x_ref[j].astype(jnp.float32)
        pooled = jnp.sum(xb, axis=-1, keepdims=True) * inv_hw
        h = jnp.dot(w1t_ref[...], pooled,
                    preferred_element_type=jnp.float32)
        h = jnp.maximum(h, 0.0)
        z = jnp.dot(w2t_ref[...], h,
                    preferred_element_type=jnp.float32)
        g = jax.nn.sigmoid(z)
        o_ref[j] = (xb * g).astype(o_ref.dtype)


def kernel(x, w1, w2):
    B, C, H, W = x.shape
    HW = H * W
    rd = w1.shape[1]

    c_pad = _round_up(C, 8)
    hw_pad = _round_up(HW, 128)
    rd_pad = _round_up(rd, 8)

    x3 = x.reshape(B, C, HW)
    if c_pad != C or hw_pad != HW:
        x3 = jnp.pad(x3, ((0, 0), (0, c_pad - C), (0, hw_pad - HW)))

    w1t = w1.astype(jnp.float32).T
    w2t = w2.astype(jnp.float32).T
    if c_pad != C or rd_pad != rd:
        w1t = jnp.pad(w1t, ((0, rd_pad - rd), (0, c_pad - C)))
        w2t = jnp.pad(w2t, ((0, c_pad - C), (0, rd_pad - rd)))

    nb = 1
    for cand in (4, 2):
        if B % cand == 0 and cand * c_pad * hw_pad * x.dtype.itemsize <= 8 * 1024 * 1024:
            nb = cand
            break

    out = pl.pallas_call(
        functools.partial(_se_kernel, inv_hw=1.0 / HW, nb=nb),
        out_shape=jax.ShapeDtypeStruct((B, c_pad, hw_pad), x.dtype),
        grid=(B // nb,),
        in_specs=[
            pl.BlockSpec((nb, c_pad, hw_pad), lambda b: (b, 0, 0)),
            pl.BlockSpec((rd_pad, c_pad), lambda b: (0, 0)),
            pl.BlockSpec((c_pad, rd_pad), lambda b: (0, 0)),
        ],
        out_specs=pl.BlockSpec((nb, c_pad, hw_pad), lambda b: (b, 0, 0)),
        compiler_params=pltpu.CompilerParams(
            dimension_semantics=("parallel",),
            vmem_limit_bytes=64 * 1024 * 1024,
        ),
    )(x3, w1t, w2t)

    if c_pad != C or hw_pad != HW:
        out = out[:, :C, :HW]
    return out.reshape(B, C, H, W)
